# baseline (speedup 1.0000x reference)
import functools

import jax
import jax.numpy as jnp
from jax import lax
from jax.experimental import pallas as pl
from jax.experimental.pallas import tpu as pltpu
from jax.experimental.pallas import tpu_sc as plsc

B, C, N, NPOINT = 8, 128, 100000, 16384
NC, NS, L = 2, 16, 16
NW = NC * NS
WPB = NW // B
CPW = C // WPB
NV = NPOINT // L

SEG = 47104
SEG2 = N // 128 * 128 - 2 * SEG
TAIL = N - N // 128 * 128
SPAN2 = SEG2 + TAIL


def _gather_kernel(feat_hbm, tails_hbm, idx_hbm, out_hbm,
                   idx_v, out_v, buf0_v, buf1_v, tails_v,
                   sem0, sem1, osem):
    wid = lax.axis_index("s") * NC + lax.axis_index("c")
    b = wid // WPB
    c0 = (wid % WPB) * CPW

    pltpu.sync_copy(idx_hbm.at[b], idx_v)
    pltpu.sync_copy(tails_hbm.at[b, pl.ds(c0 * TAIL, CPW * TAIL)], tails_v)

    bufs = (buf0_v, buf1_v)
    sems = (sem0, sem1)

    def fire_seg(r, base, size):
        bi = (r + (0 if base == 0 else 1)) % 2
        cp = pltpu.make_async_copy(
            feat_hbm.at[b, c0 + r, pl.ds(base, size)],
            bufs[bi].at[pl.ds(0, size)], sems[bi])
        cp.start()
        return cp

    def gather_pass(buf, base, span, first):
        base_v = jnp.full((L,), base, jnp.int32)
        lim_v = jnp.full((L,), span, jnp.uint32)
        cl_lim = jnp.full((L,), span - 1, jnp.uint32)
        zero_v = jnp.zeros((L,), jnp.float32)

        @plsc.parallel_loop(0, NV, step=1, unroll=8)
        def _(jl):
            iv = idx_v[pl.ds(jl * L, L)]
            loc = (iv - base_v).astype(jnp.uint32)
            inseg = loc < lim_v
            cl = jnp.minimum(loc, cl_lim).astype(jnp.int32)
            g = plsc.load_gather(buf, [cl])
            val = jnp.where(inseg, g, zero_v)
            if first:
                out_v[pl.ds(jl * L, L)] = val
            else:
                plsc.addupdate(out_v.at[pl.ds(jl * L, L)], val)

    h0 = fire_seg(0, 0, SEG)
    h1 = fire_seg(0, SEG, SEG)
    hout = None
    for r in range(CPW):
        p = r % 2
        q = 1 - p
        h0.wait()
        if hout is not None:
            hout.wait()
            hout = None
        gather_pass(bufs[p], 0, SEG, first=True)
        if r + 1 < CPW:
            h1_next = pltpu.make_async_copy(
                feat_hbm.at[b, c0 + r + 1, pl.ds(SEG, SEG)],
                bufs[p], sems[p])
            h1_next.start()
        h1.wait()
        gather_pass(bufs[q], SEG, SEG, first=False)
        h2 = pltpu.make_async_copy(
            feat_hbm.at[b, c0 + r, pl.ds(2 * SEG, SEG2)],
            bufs[q].at[pl.ds(0, SEG2)], sems[q])
        h2.start()
        h2.wait()
        tv0 = tails_v[pl.ds(r * TAIL, L)]
        tv1 = tails_v[pl.ds(r * TAIL + L, L)]
        bufs[q][pl.ds(SEG2, L)] = tv0
        bufs[q][pl.ds(SEG2 + L, L)] = tv1
        gather_pass(bufs[q], 2 * SEG, SPAN2, first=False)
        hout = pltpu.make_async_copy(out_v, out_hbm.at[b, c0 + r], osem)
        hout.start()
        if r + 1 < CPW:
            h0 = pltpu.make_async_copy(
                feat_hbm.at[b, c0 + r + 1, pl.ds(0, SEG)],
                bufs[q].at[pl.ds(0, SEG)], sems[q])
            h0.start()
            h1 = h1_next
    if hout is not None:
        hout.wait()


@jax.jit
def kernel(features, idx):
    mesh = plsc.VectorSubcoreMesh(core_axis_name="c", subcore_axis_name="s")
    tails = features[:, :, N - TAIL:].reshape(B, C * TAIL)
    run = functools.partial(
        pl.kernel,
        mesh=mesh,
        compiler_params=pltpu.CompilerParams(needs_layout_passes=False),
        out_type=jax.ShapeDtypeStruct((B, C, NPOINT), jnp.float32),
        scratch_types=[
            pltpu.VMEM((NPOINT,), jnp.int32),
            pltpu.VMEM((NPOINT,), jnp.float32),
            pltpu.VMEM((SEG,), jnp.float32),
            pltpu.VMEM((SEG,), jnp.float32),
            pltpu.VMEM((CPW * TAIL,), jnp.float32),
            pltpu.SemaphoreType.DMA,
            pltpu.SemaphoreType.DMA,
            pltpu.SemaphoreType.DMA,
        ],
    )(_gather_kernel)
    return run(features, tails, idx)


# speedup vs baseline: 1.2014x; 1.2014x over previous
import functools

import jax
import jax.numpy as jnp
from jax import lax
from jax.experimental import pallas as pl
from jax.experimental.pallas import tpu as pltpu
from jax.experimental.pallas import tpu_sc as plsc

B, C, N, NPOINT = 8, 128, 100000, 16384
NC, NS, L = 2, 16, 16
NW = NC * NS
WPB = NW // B
CPW = C // WPB
CHUNK = 4096
NCHUNK = NPOINT // CHUNK
VPC = CHUNK // L


def _gather_kernel(feat_hbm, idx_hbm, out_hbm, idx_v, row_v, obuf_v,
                   sem0, sem1, row_sem):
    wid = lax.axis_index("s") * NC + lax.axis_index("c")
    b = wid // WPB
    c0 = (wid % WPB) * CPW

    pltpu.sync_copy(idx_hbm.at[b], idx_v)

    sems = (sem0, sem1)
    pending = [None, None]
    NSPLIT = 4
    SEG = N // NSPLIT
    for ci in range(CPW):
        c = c0 + ci
        cp = pltpu.make_async_copy(feat_hbm.at[b, c], row_v, row_sem)
        cp.start()
        cp.wait()
        for t in range(NCHUNK):
            sl = t % 2
            if pending[sl] is not None:
                pending[sl].wait()
                pending[sl] = None

            @plsc.parallel_loop(0, VPC, step=1, unroll=8)
            def body(jl, t=t, sl=sl):
                iv = idx_v[pl.ds(t * CHUNK + jl * L, L)]
                g = plsc.load_gather(row_v, [iv])
                obuf_v[sl, pl.ds(jl * L, L)] = g
            cp = pltpu.make_async_copy(
                obuf_v.at[sl], out_hbm.at[b, c, pl.ds(t * CHUNK, CHUNK)],
                sems[sl])
            cp.start()
            pending[sl] = cp
    for sl in range(2):
        if pending[sl] is not None:
            pending[sl].wait()


@jax.jit
def kernel(features, idx):
    mesh = plsc.VectorSubcoreMesh(core_axis_name="c", subcore_axis_name="s")
    run = functools.partial(
        pl.kernel,
        mesh=mesh,
        compiler_params=pltpu.CompilerParams(needs_layout_passes=False),
        out_type=jax.ShapeDtypeStruct((B, C, NPOINT), jnp.float32),
        scratch_types=[
            pltpu.VMEM((NPOINT,), jnp.int32),
            pltpu.VMEM((N,), jnp.float32),
            pltpu.VMEM((2, CHUNK), jnp.float32),
            pltpu.SemaphoreType.DMA,
            pltpu.SemaphoreType.DMA,
            pltpu.SemaphoreType.DMA,
        ],
    )(_gather_kernel)
    return run(features, idx)
